# Initial kernel scaffold; baseline (speedup 1.0000x reference)
import jax
import jax.numpy as jnp
from jax.experimental import pallas as pl


def kernel(inputs, w, b):
    raise NotImplementedError("write your pallas kernel here")


# speedup vs baseline: 1.4449x; 1.4449x over previous
import jax
import jax.numpy as jnp
from jax import lax
from jax.experimental import pallas as pl
from jax.experimental.pallas import tpu as pltpu
from jax.experimental.pallas import tpu_sc as plsc

BATCH = 16384
NUM_INPUTS = 26
NUM_CORES = 2
NUM_SUBCORES = 16
NW = NUM_CORES * NUM_SUBCORES
ROWS_PER_W = BATCH // NW
IDX_PER_W = ROWS_PER_W * NUM_INPUTS
CHUNK = 128
NCHUNK = IDX_PER_W // CHUNK
L = 16


def _sc_body(inputs_hbm, w_hbm, b_hbm, out_hbm, idx_v, vals_v, out_v, b_v, sem):
    cid = lax.axis_index("c")
    sid = lax.axis_index("s")
    wid = sid * NUM_CORES + cid

    pltpu.sync_copy(inputs_hbm.at[wid], idx_v)
    pltpu.sync_copy(b_hbm, b_v)
    pltpu.async_copy(w_hbm.at[idx_v], vals_v, sem).wait()

    bvec = b_v[...]

    def blk_body(blk, carry):
        base = pl.multiple_of(blk * L, L)
        acc = bvec
        for j in range(NUM_INPUTS):
            acc = acc + vals_v[pl.ds(j * ROWS_PER_W + base, L)]
        out_v[pl.ds(base, L)] = 1.0 / (1.0 + jnp.exp(-acc))
        return carry

    lax.fori_loop(0, ROWS_PER_W // L, blk_body, 0)
    pltpu.sync_copy(
        out_v, out_hbm.at[pl.ds(pl.multiple_of(wid * ROWS_PER_W, 512), ROWS_PER_W)]
    )


def kernel(inputs, w, b):
    inputs_r = (
        inputs.reshape(NW, ROWS_PER_W, NUM_INPUTS)
        .transpose(0, 2, 1)
        .reshape(NW, IDX_PER_W)
    )
    w_flat = w.reshape(-1)
    b16 = jnp.broadcast_to(b, (L,))
    mesh = plsc.VectorSubcoreMesh(core_axis_name="c", subcore_axis_name="s")
    f = pl.kernel(
        _sc_body,
        out_type=jax.ShapeDtypeStruct((BATCH,), jnp.float32),
        mesh=mesh,
        scratch_types=[
            pltpu.VMEM((IDX_PER_W,), jnp.int32),
            pltpu.VMEM((IDX_PER_W,), jnp.float32),
            pltpu.VMEM((ROWS_PER_W,), jnp.float32),
            pltpu.VMEM((L,), jnp.float32),
            pltpu.SemaphoreType.DMA,
        ],
    )
    return f(inputs_r, w_flat, b16)


# speedup vs baseline: 2.9494x; 2.0413x over previous
import jax
import jax.numpy as jnp
from jax import lax
from jax.experimental import pallas as pl
from jax.experimental.pallas import tpu as pltpu
from jax.experimental.pallas import tpu_sc as plsc

BATCH = 16384
NUM_INPUTS = 26
NUM_CORES = 2
NUM_SUBCORES = 16
NW = NUM_CORES * NUM_SUBCORES
ROWS_PER_W = BATCH // NW
IDX_PER_W = ROWS_PER_W * NUM_INPUTS
CHUNK = 128
NCHUNK = IDX_PER_W // CHUNK
L = 16
INPUT_TABLE = 1000000


def _sc_body(inputs_hbm, w_hbm, b_hbm, out_hbm, idx_v, vals_v, out_v, b_v, sem):
    cid = lax.axis_index("c")
    sid = lax.axis_index("s")
    wid = sid * NUM_CORES + cid

    pltpu.sync_copy(inputs_hbm.at[wid], idx_v)
    pltpu.sync_copy(b_hbm, b_v)
    pltpu.async_copy(w_hbm.at[0].at[idx_v], vals_v, sem).wait()

    bvec = b_v[...]

    def blk_body(blk, carry):
        base = pl.multiple_of(blk * L, L)
        acc = bvec
        for j in range(NUM_INPUTS):
            acc = acc + vals_v[pl.ds(j * ROWS_PER_W + base, L)]
        out_v[pl.ds(base, L)] = 1.0 / (1.0 + jnp.exp(-acc))
        return carry

    lax.fori_loop(0, ROWS_PER_W // L, blk_body, 0)
    pltpu.sync_copy(
        out_v, out_hbm.at[pl.ds(pl.multiple_of(wid * ROWS_PER_W, 512), ROWS_PER_W)]
    )


def kernel(inputs, w, b):
    inputs_r = (
        inputs.reshape(NW, ROWS_PER_W, NUM_INPUTS)
        .transpose(0, 2, 1)
        .reshape(NW, IDX_PER_W)
    )
    b16 = jnp.broadcast_to(b, (L,))
    mesh = plsc.VectorSubcoreMesh(core_axis_name="c", subcore_axis_name="s")
    f = pl.kernel(
        _sc_body,
        out_type=jax.ShapeDtypeStruct((BATCH,), jnp.float32),
        mesh=mesh,
        scratch_types=[
            pltpu.VMEM((IDX_PER_W,), jnp.int32),
            pltpu.VMEM((IDX_PER_W,), jnp.float32),
            pltpu.VMEM((ROWS_PER_W,), jnp.float32),
            pltpu.VMEM((L,), jnp.float32),
            pltpu.SemaphoreType.DMA,
        ],
    )
    return f(inputs_r, w.reshape(1, INPUT_TABLE), b16)


# speedup vs baseline: 3.4257x; 1.1615x over previous
import jax
import jax.numpy as jnp
from jax import lax
from jax.experimental import pallas as pl
from jax.experimental.pallas import tpu as pltpu
from jax.experimental.pallas import tpu_sc as plsc

BATCH = 16384
NUM_INPUTS = 26
NUM_CORES = 2
NUM_SUBCORES = 16
NW = NUM_CORES * NUM_SUBCORES
ROWS_PER_W = BATCH // NW
IDX_PER_W = ROWS_PER_W * NUM_INPUTS
CHUNK = 128
NCHUNK = IDX_PER_W // CHUNK
L = 16
INPUT_TABLE = 1000000


def _sc_body(inputs_hbm, w_hbm, b_hbm, out_hbm, table_sh, idx_v, vals_v, out_v, b_v, sem):
    cid = lax.axis_index("c")
    sid = lax.axis_index("s")
    wid = sid * NUM_CORES + cid

    idx_cp = pltpu.async_copy(inputs_hbm.at[wid], idx_v, sem)
    pltpu.sync_copy(b_hbm, b_v)

    @pl.when(sid == 0)
    def _():
        pltpu.sync_copy(w_hbm, table_sh)

    plsc.subcore_barrier()
    idx_cp.wait()
    pltpu.async_copy(table_sh.at[0].at[idx_v], vals_v, sem).wait()

    bvec = b_v[...]

    def blk_body(blk, carry):
        base = pl.multiple_of(blk * L, L)
        acc = bvec
        for j in range(NUM_INPUTS):
            acc = acc + vals_v[pl.ds(j * ROWS_PER_W + base, L)]
        out_v[pl.ds(base, L)] = 1.0 / (1.0 + jnp.exp(-acc))
        return carry

    lax.fori_loop(0, ROWS_PER_W // L, blk_body, 0)
    pltpu.sync_copy(
        out_v, out_hbm.at[pl.ds(pl.multiple_of(wid * ROWS_PER_W, 512), ROWS_PER_W)]
    )


def kernel(inputs, w, b):
    inputs_r = (
        inputs.reshape(NW, ROWS_PER_W, NUM_INPUTS)
        .transpose(0, 2, 1)
        .reshape(NW, IDX_PER_W)
    )
    b16 = jnp.broadcast_to(b, (L,))
    mesh = plsc.VectorSubcoreMesh(core_axis_name="c", subcore_axis_name="s")
    f = pl.kernel(
        _sc_body,
        out_type=jax.ShapeDtypeStruct((BATCH,), jnp.float32),
        mesh=mesh,
        scratch_types=[
            pltpu.VMEM_SHARED((1, INPUT_TABLE), jnp.float32),
            pltpu.VMEM((IDX_PER_W,), jnp.int32),
            pltpu.VMEM((IDX_PER_W,), jnp.float32),
            pltpu.VMEM((ROWS_PER_W,), jnp.float32),
            pltpu.VMEM((L,), jnp.float32),
            pltpu.SemaphoreType.DMA,
        ],
    )
    return f(inputs_r, w.reshape(1, INPUT_TABLE), b16)


# speedup vs baseline: 3.4508x; 1.0073x over previous
import jax
import jax.numpy as jnp
from jax import lax
from jax.experimental import pallas as pl
from jax.experimental.pallas import tpu as pltpu
from jax.experimental.pallas import tpu_sc as plsc

BATCH = 16384
NUM_INPUTS = 26
NUM_CORES = 2
NUM_SUBCORES = 16
NW = NUM_CORES * NUM_SUBCORES
ROWS_PER_W = BATCH // NW
IDX_PER_W = ROWS_PER_W * NUM_INPUTS
CHUNK = 128
NCHUNK = IDX_PER_W // CHUNK
L = 16
INPUT_TABLE = 1000000


def _sc_body(inputs_hbm, w_hbm, b_hbm, out_hbm, table_sh, idx_v, vals_v, out_v, b_v, sem):
    cid = lax.axis_index("c")
    sid = lax.axis_index("s")
    wid = sid * NUM_CORES + cid

    idx_cp = pltpu.async_copy(inputs_hbm.at[wid], idx_v, sem)
    pltpu.sync_copy(b_hbm, b_v)

    for t in range(4):
        @pl.when(sid == t)
        def _():
            off = 250112 * t
            ln = 250112 if t < 3 else INPUT_TABLE - 3 * 250112
            pltpu.sync_copy(
                w_hbm.at[:, pl.ds(off, ln)], table_sh.at[:, pl.ds(off, ln)]
            )

    plsc.subcore_barrier()
    idx_cp.wait()
    pltpu.async_copy(table_sh.at[0].at[idx_v], vals_v, sem).wait()

    bvec = b_v[...]

    def blk_body(blk, carry):
        base = pl.multiple_of(blk * L, L)
        acc = bvec
        for j in range(NUM_INPUTS):
            acc = acc + vals_v[pl.ds(j * ROWS_PER_W + base, L)]
        out_v[pl.ds(base, L)] = 1.0 / (1.0 + jnp.exp(-acc))
        return carry

    lax.fori_loop(0, ROWS_PER_W // L, blk_body, 0)
    pltpu.sync_copy(
        out_v, out_hbm.at[pl.ds(pl.multiple_of(wid * ROWS_PER_W, 512), ROWS_PER_W)]
    )


def kernel(inputs, w, b):
    inputs_r = (
        inputs.reshape(NW, ROWS_PER_W, NUM_INPUTS)
        .transpose(0, 2, 1)
        .reshape(NW, IDX_PER_W)
    )
    b16 = jnp.broadcast_to(b, (L,))
    mesh = plsc.VectorSubcoreMesh(core_axis_name="c", subcore_axis_name="s")
    f = pl.kernel(
        _sc_body,
        out_type=jax.ShapeDtypeStruct((BATCH,), jnp.float32),
        mesh=mesh,
        scratch_types=[
            pltpu.VMEM_SHARED((1, INPUT_TABLE), jnp.float32),
            pltpu.VMEM((IDX_PER_W,), jnp.int32),
            pltpu.VMEM((IDX_PER_W,), jnp.float32),
            pltpu.VMEM((ROWS_PER_W,), jnp.float32),
            pltpu.VMEM((L,), jnp.float32),
            pltpu.SemaphoreType.DMA,
        ],
    )
    return f(inputs_r, w.reshape(1, INPUT_TABLE), b16)


# speedup vs baseline: 3.4647x; 1.0041x over previous
import jax
import jax.numpy as jnp
from jax import lax
from jax.experimental import pallas as pl
from jax.experimental.pallas import tpu as pltpu
from jax.experimental.pallas import tpu_sc as plsc

BATCH = 16384
NUM_INPUTS = 26
NUM_CORES = 2
NUM_SUBCORES = 16
NW = NUM_CORES * NUM_SUBCORES
ROWS_PER_W = BATCH // NW
IDX_PER_W = ROWS_PER_W * NUM_INPUTS
CHUNK = 128
NCHUNK = IDX_PER_W // CHUNK
L = 16
INPUT_TABLE = 1000000


def _sc_body(inputs_hbm, w_hbm, b_hbm, out_hbm, table_sh, idx_v, vals_v, out_v, b_v, sem):
    cid = lax.axis_index("c")
    sid = lax.axis_index("s")
    wid = sid * NUM_CORES + cid

    with jax.named_scope("p_idx_issue"):
        idx_cp = pltpu.async_copy(inputs_hbm.at[wid], idx_v, sem)
        pltpu.sync_copy(b_hbm, b_v)

    with jax.named_scope("p_table_load"):
        for t in range(4):
            @pl.when(sid == t)
            def _():
                off = 250112 * t
                ln = 250112 if t < 3 else INPUT_TABLE - 3 * 250112
                pltpu.sync_copy(
                    w_hbm.at[:, pl.ds(off, ln)], table_sh.at[:, pl.ds(off, ln)]
                )

    with jax.named_scope("p_barrier"):
        plsc.subcore_barrier()
        idx_cp.wait()
    with jax.named_scope("p_gather"):
        pltpu.async_copy(table_sh.at[0].at[idx_v], vals_v, sem).wait()

    bvec = b_v[...]

    def blk_body(blk, carry):
        base = pl.multiple_of(blk * L, L)
        acc = bvec
        for j in range(NUM_INPUTS):
            acc = acc + vals_v[pl.ds(j * ROWS_PER_W + base, L)]
        out_v[pl.ds(base, L)] = 1.0 / (1.0 + jnp.exp(-acc))
        return carry

    with jax.named_scope("p_reduce"):
        lax.fori_loop(0, ROWS_PER_W // L, blk_body, 0)
    pltpu.sync_copy(
        out_v, out_hbm.at[pl.ds(pl.multiple_of(wid * ROWS_PER_W, 512), ROWS_PER_W)]
    )


def kernel(inputs, w, b):
    inputs_r = (
        inputs.reshape(NW, ROWS_PER_W, NUM_INPUTS)
        .transpose(0, 2, 1)
        .reshape(NW, IDX_PER_W)
    )
    b16 = jnp.broadcast_to(b, (L,))
    mesh = plsc.VectorSubcoreMesh(core_axis_name="c", subcore_axis_name="s")
    f = pl.kernel(
        _sc_body,
        out_type=jax.ShapeDtypeStruct((BATCH,), jnp.float32),
        mesh=mesh,
        scratch_types=[
            pltpu.VMEM_SHARED((1, INPUT_TABLE), jnp.float32),
            pltpu.VMEM((IDX_PER_W,), jnp.int32),
            pltpu.VMEM((IDX_PER_W,), jnp.float32),
            pltpu.VMEM((ROWS_PER_W,), jnp.float32),
            pltpu.VMEM((L,), jnp.float32),
            pltpu.SemaphoreType.DMA,
        ],
    )
    return f(inputs_r, w.reshape(1, INPUT_TABLE), b16)


# speedup vs baseline: 3.5453x; 1.0232x over previous
import jax
import jax.numpy as jnp
from jax import lax
from jax.experimental import pallas as pl
from jax.experimental.pallas import tpu as pltpu
from jax.experimental.pallas import tpu_sc as plsc

BATCH = 16384
NUM_INPUTS = 26
NUM_CORES = 2
NUM_SUBCORES = 16
NW = NUM_CORES * NUM_SUBCORES
ROWS_PER_W = BATCH // NW
IDX_PER_W = ROWS_PER_W * NUM_INPUTS
CHUNK = 128
NCHUNK = IDX_PER_W // CHUNK
L = 16
INPUT_TABLE = 1000000


def _sc_body(inputs_hbm, w_hbm, b_hbm, out_hbm, table_sh, idx_v, vals_v, out_v, b_v, sem):
    cid = lax.axis_index("c")
    sid = lax.axis_index("s")
    wid = sid * NUM_CORES + cid

    idx_cp = pltpu.async_copy(inputs_hbm.at[wid], idx_v, sem)
    pltpu.sync_copy(b_hbm, b_v)

    CH = 125056
    for t in range(8):
        @pl.when(sid == t)
        def _():
            off = CH * t
            ln = CH if t < 7 else INPUT_TABLE - 7 * CH
            pltpu.sync_copy(
                w_hbm.at[:, pl.ds(off, ln)], table_sh.at[:, pl.ds(off, ln)]
            )

    plsc.subcore_barrier()
    idx_cp.wait()
    pltpu.async_copy(table_sh.at[0].at[idx_v], vals_v, sem).wait()

    bvec = b_v[...]

    def blk_body(blk, carry):
        base = pl.multiple_of(blk * L, L)
        acc = bvec
        for j in range(NUM_INPUTS):
            acc = acc + vals_v[pl.ds(j * ROWS_PER_W + base, L)]
        out_v[pl.ds(base, L)] = 1.0 / (1.0 + jnp.exp(-acc))
        return carry

    lax.fori_loop(0, ROWS_PER_W // L, blk_body, 0)
    pltpu.sync_copy(
        out_v, out_hbm.at[pl.ds(pl.multiple_of(wid * ROWS_PER_W, 512), ROWS_PER_W)]
    )


def kernel(inputs, w, b):
    inputs_r = (
        inputs.reshape(NW, ROWS_PER_W, NUM_INPUTS)
        .transpose(0, 2, 1)
        .reshape(NW, IDX_PER_W)
    )
    b16 = jnp.broadcast_to(b, (L,))
    mesh = plsc.VectorSubcoreMesh(core_axis_name="c", subcore_axis_name="s")
    f = pl.kernel(
        _sc_body,
        out_type=jax.ShapeDtypeStruct((BATCH,), jnp.float32),
        mesh=mesh,
        scratch_types=[
            pltpu.VMEM_SHARED((1, INPUT_TABLE), jnp.float32),
            pltpu.VMEM((IDX_PER_W,), jnp.int32),
            pltpu.VMEM((IDX_PER_W,), jnp.float32),
            pltpu.VMEM((ROWS_PER_W,), jnp.float32),
            pltpu.VMEM((L,), jnp.float32),
            pltpu.SemaphoreType.DMA,
        ],
    )
    return f(inputs_r, w.reshape(1, INPUT_TABLE), b16)


# speedup vs baseline: 3.5844x; 1.0110x over previous
import jax
import jax.numpy as jnp
from jax import lax
from jax.experimental import pallas as pl
from jax.experimental.pallas import tpu as pltpu
from jax.experimental.pallas import tpu_sc as plsc

BATCH = 16384
NUM_INPUTS = 26
NUM_CORES = 2
NUM_SUBCORES = 16
NW = NUM_CORES * NUM_SUBCORES
ROWS_PER_W = BATCH // NW
IDX_PER_W = ROWS_PER_W * NUM_INPUTS
CHUNK = 128
NCHUNK = IDX_PER_W // CHUNK
L = 16
INPUT_TABLE = 1000000


def _sc_body(inputs_hbm, w_hbm, b_hbm, out_hbm, table_sh, idx_v, vals_v, out_v, b_v, sem):
    cid = lax.axis_index("c")
    sid = lax.axis_index("s")
    wid = sid * NUM_CORES + cid

    idx_cp = pltpu.async_copy(inputs_hbm.at[wid], idx_v, sem)
    pltpu.sync_copy(b_hbm, b_v)

    CH = 62592
    for t in range(16):
        @pl.when(sid == t)
        def _():
            off = CH * t
            ln = CH if t < 15 else INPUT_TABLE - 15 * CH
            pltpu.sync_copy(
                w_hbm.at[:, pl.ds(off, ln)], table_sh.at[:, pl.ds(off, ln)]
            )

    plsc.subcore_barrier()
    idx_cp.wait()
    pltpu.async_copy(table_sh.at[0].at[idx_v], vals_v, sem).wait()

    bvec = b_v[...]

    def blk_body(blk, carry):
        base = pl.multiple_of(blk * L, L)
        acc = bvec
        for j in range(NUM_INPUTS):
            acc = acc + vals_v[pl.ds(j * ROWS_PER_W + base, L)]
        out_v[pl.ds(base, L)] = 1.0 / (1.0 + jnp.exp(-acc))
        return carry

    lax.fori_loop(0, ROWS_PER_W // L, blk_body, 0)
    pltpu.sync_copy(
        out_v, out_hbm.at[pl.ds(pl.multiple_of(wid * ROWS_PER_W, 512), ROWS_PER_W)]
    )


def kernel(inputs, w, b):
    inputs_r = (
        inputs.reshape(NW, ROWS_PER_W, NUM_INPUTS)
        .transpose(0, 2, 1)
        .reshape(NW, IDX_PER_W)
    )
    b16 = jnp.broadcast_to(b, (L,))
    mesh = plsc.VectorSubcoreMesh(core_axis_name="c", subcore_axis_name="s")
    f = pl.kernel(
        _sc_body,
        out_type=jax.ShapeDtypeStruct((BATCH,), jnp.float32),
        mesh=mesh,
        scratch_types=[
            pltpu.VMEM_SHARED((1, INPUT_TABLE), jnp.float32),
            pltpu.VMEM((IDX_PER_W,), jnp.int32),
            pltpu.VMEM((IDX_PER_W,), jnp.float32),
            pltpu.VMEM((ROWS_PER_W,), jnp.float32),
            pltpu.VMEM((L,), jnp.float32),
            pltpu.SemaphoreType.DMA,
        ],
    )
    return f(inputs_r, w.reshape(1, INPUT_TABLE), b16)


# speedup vs baseline: 3.6027x; 1.0051x over previous
import jax
import jax.numpy as jnp
from jax import lax
from jax.experimental import pallas as pl
from jax.experimental.pallas import tpu as pltpu
from jax.experimental.pallas import tpu_sc as plsc

BATCH = 16384
NUM_INPUTS = 26
NUM_CORES = 2
NUM_SUBCORES = 16
NW = NUM_CORES * NUM_SUBCORES
ROWS_PER_W = BATCH // NW
IDX_PER_W = ROWS_PER_W * NUM_INPUTS
L = 16
TABLE = 1000000
LOAD_CHUNK = 62592


def _sc_body(inputs_hbm, w_hbm, b_hbm, out_hbm, table_sh, idx_v, vals_v, out_v, b_v, sem):
    cid = lax.axis_index("c")
    sid = lax.axis_index("s")
    wid = sid * NUM_CORES + cid

    idx_cp = pltpu.async_copy(inputs_hbm.at[wid], idx_v, sem)
    pltpu.sync_copy(b_hbm, b_v)

    for t in range(NUM_SUBCORES):
        @pl.when(sid == t)
        def _():
            off = LOAD_CHUNK * t
            ln = LOAD_CHUNK if t < 15 else TABLE - 15 * LOAD_CHUNK
            pltpu.sync_copy(
                w_hbm.at[:, pl.ds(off, ln)], table_sh.at[:, pl.ds(off, ln)]
            )

    plsc.subcore_barrier()
    idx_cp.wait()
    pltpu.async_copy(table_sh.at[0].at[idx_v], vals_v, sem).wait()

    bvec = b_v[...]

    def blk_body(blk, carry):
        base = pl.multiple_of(blk * L, L)
        acc = bvec
        for j in range(NUM_INPUTS):
            acc = acc + vals_v[pl.ds(j * ROWS_PER_W + base, L)]
        out_v[pl.ds(base, L)] = 1.0 / (1.0 + jnp.exp(-acc))
        return carry

    lax.fori_loop(0, ROWS_PER_W // L, blk_body, 0)
    pltpu.sync_copy(
        out_v, out_hbm.at[pl.ds(pl.multiple_of(wid * ROWS_PER_W, 512), ROWS_PER_W)]
    )


def kernel(inputs, w, b):
    inputs_r = (
        inputs.reshape(NW, ROWS_PER_W, NUM_INPUTS)
        .transpose(0, 2, 1)
        .reshape(NW, IDX_PER_W)
    )
    b16 = jnp.broadcast_to(b, (L,))
    mesh = plsc.VectorSubcoreMesh(core_axis_name="c", subcore_axis_name="s")
    f = pl.kernel(
        _sc_body,
        out_type=jax.ShapeDtypeStruct((BATCH,), jnp.float32),
        mesh=mesh,
        scratch_types=[
            pltpu.VMEM_SHARED((1, TABLE), jnp.float32),
            pltpu.VMEM((IDX_PER_W,), jnp.int32),
            pltpu.VMEM((IDX_PER_W,), jnp.float32),
            pltpu.VMEM((ROWS_PER_W,), jnp.float32),
            pltpu.VMEM((L,), jnp.float32),
            pltpu.SemaphoreType.DMA,
        ],
    )
    return f(inputs_r, w.reshape(1, TABLE), b16)
